# baseline (speedup 1.0000x reference)
import dataclasses

import jax
import jax.numpy as jnp
from jax import lax
from jax.experimental import pallas as pl
from jax.experimental.pallas import tpu as pltpu
from jax.experimental.pallas import tpu_sc as plsc

_B, _N, _F, _H, _OUT, _K = 32, 256, 512, 512, 8, 5
_NBR_PAD = 16
_SC_L = 16


def _masked_dists(g, mvec):
    cols = []
    for j in range(_B):
        diff = g - g[j:j + 1, :]
        cols.append(jnp.sum(diff * diff, axis=1, keepdims=True))
    d = jnp.concatenate(cols, axis=1)
    rows = lax.broadcasted_iota(jnp.int32, (_B, _B), 0)
    cols = lax.broadcasted_iota(jnp.int32, (_B, _B), 1)
    colsf = cols.astype(jnp.float32)
    bad = (rows == cols) | (jnp.broadcast_to(mvec, (_B, _B)) != 0)
    return jnp.where(bad, 1e30 + colsf * 1e24, d)


def _sc_topk_body(dm1_hbm, dm2_hbm, nbr1_hbm, nbr2_hbm, row_v, out_v, sem):
    wid = lax.axis_index("s") * 2 + lax.axis_index("c")
    i1 = lax.iota(jnp.int32, 16)
    i2 = i1 + 16
    for dm_hbm, nbr_hbm in ((dm1_hbm, nbr1_hbm), (dm2_hbm, nbr2_hbm)):
        pltpu.async_copy(dm_hbm.at[wid], row_v, sem).wait()
        v1 = row_v[pl.ds(0, _SC_L)]
        v2 = row_v[pl.ds(_SC_L, _SC_L)]
        outv = jnp.zeros((_SC_L,), jnp.int32)
        for k in range(_K):
            ms = jnp.minimum(jnp.min(v1), jnp.min(v2))
            c1 = jnp.min(jnp.where(v1 == ms, i1, 2 * _B))
            c2 = jnp.min(jnp.where(v2 == ms, i2, 2 * _B))
            idx = jnp.minimum(c1, c2)
            outv = jnp.where(i1 == k, idx, outv)
            v1 = jnp.where(i1 == idx, jnp.float32(jnp.inf), v1)
            v2 = jnp.where(i2 == idx, jnp.float32(jnp.inf), v2)
        out_v[...] = outv
        pltpu.async_copy(out_v, nbr_hbm.at[wid], sem).wait()


_G = 4


def _gcn_body(A_ref, x_ref,
              wsc0, bsc0, wsc1, bsc1, wsc2, bsc2,
              wfc0, bfc0, wfc1, bfc1, wfc2, bfc2,
              m1_ref, m2_ref,
              es_ref, ef_ref, dm1_ref, dm2_ref,
              gs_ref, gf_ref):
    i = pl.program_id(0)

    def layer(x, A, W, b):
        xw = jnp.dot(x, W[...], preferred_element_type=jnp.float32)
        y = jnp.dot(A, xw, preferred_element_type=jnp.float32) + b[...]
        return jnp.maximum(y, 0.0)

    weights = ((wsc0, bsc0, wfc0, bfc0),
               (wsc1, bsc1, wfc1, bfc1),
               (wsc2, bsc2, wfc2, bfc2))
    xs = [x_ref[g] for g in range(_G)]
    xf = [x_ref[g] for g in range(_G)]
    As = [A_ref[g, 0] for g in range(_G)]
    Af = [A_ref[g, 1] for g in range(_G)]
    accs = [None] * _G
    accf = [None] * _G
    for (Ws, bs, Wf, bf) in weights:
        for g in range(_G):
            xs[g] = layer(xs[g], As[g], Ws, bs)
            xf[g] = layer(xf[g], Af[g], Wf, bf)
            accs[g] = xs[g] if accs[g] is None else accs[g] + xs[g]
            accf[g] = xf[g] if accf[g] is None else accf[g] + xf[g]
    for g in range(_G):
        a_s = accs[g] / 3.0
        a_f = accf[g] / 3.0
        es_ref[g] = a_s.astype(jnp.bfloat16)
        ef_ref[g] = a_f.astype(jnp.bfloat16)
        gs_ref[pl.ds(i * _G + g, 1), :] = jnp.mean(a_s, axis=0, keepdims=True)
        gf_ref[pl.ds(i * _G + g, 1), :] = jnp.mean(a_f, axis=0, keepdims=True)

    @pl.when(i == _B // _G - 1)
    def _():
        dm1_ref[...] = _masked_dists(gf_ref[...], m1_ref[...])
        dm2_ref[...] = _masked_dists(gs_ref[...], m2_ref[...])


_G3 = 2


def _fuse_body(n1, n2, m1, m2, *args):
    sblk = args[0:_G3 * _K]
    fblk = args[_G3 * _K:2 * _G3 * _K]
    w1w, w2w, w1b, w2b, att, outw, outb = args[2 * _G3 * _K:2 * _G3 * _K + 7]
    o_ref = args[2 * _G3 * _K + 7]
    w1a_s, w2a_s, c1_s, c2_s = args[2 * _G3 * _K + 8:2 * _G3 * _K + 12]
    b = pl.program_id(0)

    @pl.when(b == 0)
    def _():
        w1a_s[...] = jnp.dot(w1w[...], att[...],
                             preferred_element_type=jnp.float32)
        w2a_s[...] = jnp.dot(w2w[...], att[...],
                             preferred_element_type=jnp.float32)
        c1_s[...] = jnp.dot(w1b[...], att[...],
                            preferred_element_type=jnp.float32)
        c2_s[...] = jnp.dot(w2b[...], att[...],
                            preferred_element_type=jnp.float32)

    f32 = jnp.float32
    for g in range(_G3):
        e1 = sblk[g * _K][0].astype(f32)
        e2 = fblk[g * _K][0].astype(f32)
        for j in range(1, _K):
            e1 = e1 + sblk[g * _K + j][0].astype(f32)
            e2 = e2 + fblk[g * _K + j][0].astype(f32)
        e1 = e1 * (1.0 / _K)
        e2 = e2 * (1.0 / _K)
        s1 = jnp.dot(e1, w1a_s[...], preferred_element_type=f32) + c1_s[0, 0]
        s2 = jnp.dot(e2, w2a_s[...], preferred_element_type=f32) + c2_s[0, 0]
        s1 = jnp.where(s1 >= 0.0, s1, 0.3 * s1)
        s2 = jnp.where(s2 >= 0.0, s2, 0.3 * s2)
        mx = jnp.maximum(s1, s2)
        x1 = jnp.exp(s1 - mx)
        x2 = jnp.exp(s2 - mx)
        tot = x1 + x2
        comb = (x1 / tot) * e1 + (x2 / tot) * e2
        pooled = jnp.mean(comb, axis=0, keepdims=True)
        row = jnp.dot(pooled, outw[...],
                      preferred_element_type=f32) + outb[...]
        o_ref[pl.ds(b * _G3 + g, 1), :] = row


def kernel(A_batch, feature, no_sc_idx, no_fc_idx,
           W_sc0, b_sc0, W_sc1, b_sc1, W_sc2, b_sc2,
           W_fc0, b_fc0, W_fc1, b_fc1, W_fc2, b_fc2,
           w1_w, w1_b, w2_w, w2_b, attention, out_w, out_b):
    f32 = jnp.float32
    m1 = no_sc_idx.astype(jnp.int32).reshape(1, _B)
    m2 = no_fc_idx.astype(jnp.int32).reshape(1, _B)
    b2 = lambda b: b.reshape(1, -1).astype(f32)

    wfull = lambda shp: pl.BlockSpec(shp, lambda i: (0,) * len(shp))
    es, ef, dm1, dm2 = pl.pallas_call(
        _gcn_body,
        grid=(_B // _G,),
        in_specs=[
            pl.BlockSpec((_G, 2, _N, _N), lambda i: (i, 0, 0, 0)),
            pl.BlockSpec((_G, _N, _F), lambda i: (i, 0, 0)),
            wfull((_F, _H)), wfull((1, _H)),
            wfull((_H, _H)), wfull((1, _H)),
            wfull((_H, _H)), wfull((1, _H)),
            wfull((_F, _H)), wfull((1, _H)),
            wfull((_H, _H)), wfull((1, _H)),
            wfull((_H, _H)), wfull((1, _H)),
            wfull((1, _B)), wfull((1, _B)),
        ],
        out_specs=[
            pl.BlockSpec((_G, _N, _H), lambda i: (i, 0, 0)),
            pl.BlockSpec((_G, _N, _H), lambda i: (i, 0, 0)),
            pl.BlockSpec((_B, _B), lambda i: (0, 0)),
            pl.BlockSpec((_B, _B), lambda i: (0, 0)),
        ],
        out_shape=[
            jax.ShapeDtypeStruct((_B, _N, _H), jnp.bfloat16),
            jax.ShapeDtypeStruct((_B, _N, _H), jnp.bfloat16),
            jax.ShapeDtypeStruct((_B, _B), f32),
            jax.ShapeDtypeStruct((_B, _B), f32),
        ],
        scratch_shapes=[
            pltpu.VMEM((_B, _H), f32),
            pltpu.VMEM((_B, _H), f32),
        ],
        compiler_params=pltpu.CompilerParams(
            dimension_semantics=("arbitrary",)),
    )(A_batch, feature,
      W_sc0, b2(b_sc0), W_sc1, b2(b_sc1), W_sc2, b2(b_sc2),
      W_fc0, b2(b_fc0), W_fc1, b2(b_fc1), W_fc2, b2(b_fc2),
      m1, m2)

    sc_cp = pltpu.CompilerParams()
    if "needs_layout_passes" in pltpu.CompilerParams.__dataclass_fields__:
        sc_cp = dataclasses.replace(sc_cp, needs_layout_passes=False)
    sc_mesh = plsc.VectorSubcoreMesh(core_axis_name="c", subcore_axis_name="s")
    sc_topk = pl.kernel(
        _sc_topk_body,
        out_type=[
            jax.ShapeDtypeStruct((_B, _NBR_PAD), jnp.int32),
            jax.ShapeDtypeStruct((_B, _NBR_PAD), jnp.int32),
        ],
        mesh=sc_mesh,
        scratch_types=[
            pltpu.VMEM((_B,), f32),
            pltpu.VMEM((_NBR_PAD,), jnp.int32),
            pltpu.SemaphoreType.DMA,
        ],
        compiler_params=sc_cp,
    )
    nbr1, nbr2 = sc_topk(dm1, dm2)

    m1v = no_sc_idx.astype(jnp.int32)
    m2v = no_fc_idx.astype(jnp.int32)

    def nbr_map(j, branch, g):
        if branch == 1:
            return lambda b, n1, n2, mm1, mm2: (
                jnp.where(mm1[b * _G3 + g] != 0,
                          n1[b * _G3 + g, j], b * _G3 + g), 0, 0)
        return lambda b, n1, n2, mm1, mm2: (
            jnp.where(mm2[b * _G3 + g] != 0,
                      n2[b * _G3 + g, j], b * _G3 + g), 0, 0)

    grid_spec = pltpu.PrefetchScalarGridSpec(
        num_scalar_prefetch=4,
        grid=(_B // _G3,),
        in_specs=[
            *[pl.BlockSpec((1, _N, _H), nbr_map(j, 1, g))
              for g in range(_G3) for j in range(_K)],
            *[pl.BlockSpec((1, _N, _H), nbr_map(j, 2, g))
              for g in range(_G3) for j in range(_K)],
            pl.BlockSpec((_H, _H), lambda *_: (0, 0)),
            pl.BlockSpec((_H, _H), lambda *_: (0, 0)),
            pl.BlockSpec((1, _H), lambda *_: (0, 0)),
            pl.BlockSpec((1, _H), lambda *_: (0, 0)),
            pl.BlockSpec((_H, 1), lambda *_: (0, 0)),
            pl.BlockSpec((_H, _OUT), lambda *_: (0, 0)),
            pl.BlockSpec((1, _OUT), lambda *_: (0, 0)),
        ],
        out_specs=pl.BlockSpec((_B, _OUT), lambda b, *_: (0, 0)),
        scratch_shapes=[
            pltpu.VMEM((_H, 1), f32),
            pltpu.VMEM((_H, 1), f32),
            pltpu.VMEM((1, 1), f32),
            pltpu.VMEM((1, 1), f32),
        ],
    )
    out = pl.pallas_call(
        _fuse_body,
        grid_spec=grid_spec,
        out_shape=jax.ShapeDtypeStruct((_B, _OUT), f32),
        compiler_params=pltpu.CompilerParams(
            dimension_semantics=("arbitrary",)),
    )(nbr1, nbr2, m1v, m2v,
      *([es] * (_G3 * _K)), *([ef] * (_G3 * _K)),
      w1_w, w2_w, b2(w1_b), b2(w2_b), attention, out_w, b2(out_b))
    return out


# speedup vs baseline: 1.0369x; 1.0369x over previous
import dataclasses

import jax
import jax.numpy as jnp
from jax import lax
from jax.experimental import pallas as pl
from jax.experimental.pallas import tpu as pltpu
from jax.experimental.pallas import tpu_sc as plsc

_B, _N, _F, _H, _OUT, _K = 32, 256, 512, 512, 8, 5
_NBR_PAD = 16
_SC_L = 16


def _masked_dists(g, mvec):
    cols = []
    for j in range(_B):
        diff = g - g[j:j + 1, :]
        cols.append(jnp.sum(diff * diff, axis=1, keepdims=True))
    d = jnp.concatenate(cols, axis=1)
    rows = lax.broadcasted_iota(jnp.int32, (_B, _B), 0)
    cols = lax.broadcasted_iota(jnp.int32, (_B, _B), 1)
    colsf = cols.astype(jnp.float32)
    bad = (rows == cols) | (jnp.broadcast_to(mvec, (_B, _B)) != 0)
    return jnp.where(bad, 1e30 + colsf * 1e24, d)


def _sc_topk_body(dm1_hbm, dm2_hbm, nbr1_hbm, nbr2_hbm, row_v, out_v, sem):
    wid = lax.axis_index("s") * 2 + lax.axis_index("c")
    i1 = lax.iota(jnp.int32, 16)
    i2 = i1 + 16
    for dm_hbm, nbr_hbm in ((dm1_hbm, nbr1_hbm), (dm2_hbm, nbr2_hbm)):
        pltpu.async_copy(dm_hbm.at[wid], row_v, sem).wait()
        v1 = row_v[pl.ds(0, _SC_L)]
        v2 = row_v[pl.ds(_SC_L, _SC_L)]
        outv = jnp.zeros((_SC_L,), jnp.int32)
        for k in range(_K):
            ms = jnp.minimum(jnp.min(v1), jnp.min(v2))
            c1 = jnp.min(jnp.where(v1 == ms, i1, 2 * _B))
            c2 = jnp.min(jnp.where(v2 == ms, i2, 2 * _B))
            idx = jnp.minimum(c1, c2)
            outv = jnp.where(i1 == k, idx, outv)
            v1 = jnp.where(i1 == idx, jnp.float32(jnp.inf), v1)
            v2 = jnp.where(i2 == idx, jnp.float32(jnp.inf), v2)
        out_v[...] = outv
        pltpu.async_copy(out_v, nbr_hbm.at[wid], sem).wait()


_G = 4


def _gcn_body(A_ref, x_ref,
              wsc0, bsc0, wsc1, bsc1, wsc2, bsc2,
              wfc0, bfc0, wfc1, bfc1, wfc2, bfc2,
              m1_ref, m2_ref,
              es_ref, ef_ref, dm1_ref, dm2_ref,
              gs_ref, gf_ref):
    i = pl.program_id(0)

    def layer(x, A, W, b):
        xw = jnp.dot(x, W[...], preferred_element_type=jnp.float32)
        y = jnp.dot(A, xw, preferred_element_type=jnp.float32) + b[...]
        return jnp.maximum(y, 0.0)

    weights = ((wsc0, bsc0, wfc0, bfc0),
               (wsc1, bsc1, wfc1, bfc1),
               (wsc2, bsc2, wfc2, bfc2))
    xs = [x_ref[g] for g in range(_G)]
    xf = [x_ref[g] for g in range(_G)]
    As = [A_ref[g, 0] for g in range(_G)]
    Af = [A_ref[g, 1] for g in range(_G)]
    accs = [None] * _G
    accf = [None] * _G
    for (Ws, bs, Wf, bf) in weights:
        for g in range(_G):
            xs[g] = layer(xs[g], As[g], Ws, bs)
            xf[g] = layer(xf[g], Af[g], Wf, bf)
            accs[g] = xs[g] if accs[g] is None else accs[g] + xs[g]
            accf[g] = xf[g] if accf[g] is None else accf[g] + xf[g]
    for g in range(_G):
        a_s = accs[g] / 3.0
        a_f = accf[g] / 3.0
        es_ref[g] = a_s.astype(jnp.bfloat16)
        ef_ref[g] = a_f.astype(jnp.bfloat16)
        gs_ref[pl.ds(i * _G + g, 1), :] = jnp.mean(a_s, axis=0, keepdims=True)
        gf_ref[pl.ds(i * _G + g, 1), :] = jnp.mean(a_f, axis=0, keepdims=True)

    @pl.when(i == _B // _G - 1)
    def _():
        dm1_ref[...] = _masked_dists(gf_ref[...], m1_ref[...])
        dm2_ref[...] = _masked_dists(gs_ref[...], m2_ref[...])


_G3 = 4


def _fuse_body(n1, n2, m1, m2, *args):
    sblk = args[0:_G3 * _K]
    fblk = args[_G3 * _K:2 * _G3 * _K]
    w1w, w2w, w1b, w2b, att, outw, outb = args[2 * _G3 * _K:2 * _G3 * _K + 7]
    o_ref = args[2 * _G3 * _K + 7]
    w1a_s, w2a_s, c1_s, c2_s = args[2 * _G3 * _K + 8:2 * _G3 * _K + 12]
    b = pl.program_id(0)

    @pl.when(b == 0)
    def _():
        w1a_s[...] = jnp.dot(w1w[...], att[...],
                             preferred_element_type=jnp.float32)
        w2a_s[...] = jnp.dot(w2w[...], att[...],
                             preferred_element_type=jnp.float32)
        c1_s[...] = jnp.dot(w1b[...], att[...],
                            preferred_element_type=jnp.float32)
        c2_s[...] = jnp.dot(w2b[...], att[...],
                            preferred_element_type=jnp.float32)

    f32 = jnp.float32
    for g in range(_G3):
        e1 = sblk[g * _K][0].astype(f32)
        e2 = fblk[g * _K][0].astype(f32)
        for j in range(1, _K):
            e1 = e1 + sblk[g * _K + j][0].astype(f32)
            e2 = e2 + fblk[g * _K + j][0].astype(f32)
        e1 = e1 * (1.0 / _K)
        e2 = e2 * (1.0 / _K)
        s1 = jnp.dot(e1, w1a_s[...], preferred_element_type=f32) + c1_s[0, 0]
        s2 = jnp.dot(e2, w2a_s[...], preferred_element_type=f32) + c2_s[0, 0]
        s1 = jnp.where(s1 >= 0.0, s1, 0.3 * s1)
        s2 = jnp.where(s2 >= 0.0, s2, 0.3 * s2)
        mx = jnp.maximum(s1, s2)
        x1 = jnp.exp(s1 - mx)
        x2 = jnp.exp(s2 - mx)
        tot = x1 + x2
        comb = (x1 / tot) * e1 + (x2 / tot) * e2
        pooled = jnp.mean(comb, axis=0, keepdims=True)
        row = jnp.dot(pooled, outw[...],
                      preferred_element_type=f32) + outb[...]
        o_ref[pl.ds(b * _G3 + g, 1), :] = row


def kernel(A_batch, feature, no_sc_idx, no_fc_idx,
           W_sc0, b_sc0, W_sc1, b_sc1, W_sc2, b_sc2,
           W_fc0, b_fc0, W_fc1, b_fc1, W_fc2, b_fc2,
           w1_w, w1_b, w2_w, w2_b, attention, out_w, out_b):
    f32 = jnp.float32
    m1 = no_sc_idx.astype(jnp.int32).reshape(1, _B)
    m2 = no_fc_idx.astype(jnp.int32).reshape(1, _B)
    b2 = lambda b: b.reshape(1, -1).astype(f32)

    wfull = lambda shp: pl.BlockSpec(shp, lambda i: (0,) * len(shp))
    es, ef, dm1, dm2 = pl.pallas_call(
        _gcn_body,
        grid=(_B // _G,),
        in_specs=[
            pl.BlockSpec((_G, 2, _N, _N), lambda i: (i, 0, 0, 0)),
            pl.BlockSpec((_G, _N, _F), lambda i: (i, 0, 0)),
            wfull((_F, _H)), wfull((1, _H)),
            wfull((_H, _H)), wfull((1, _H)),
            wfull((_H, _H)), wfull((1, _H)),
            wfull((_F, _H)), wfull((1, _H)),
            wfull((_H, _H)), wfull((1, _H)),
            wfull((_H, _H)), wfull((1, _H)),
            wfull((1, _B)), wfull((1, _B)),
        ],
        out_specs=[
            pl.BlockSpec((_G, _N, _H), lambda i: (i, 0, 0)),
            pl.BlockSpec((_G, _N, _H), lambda i: (i, 0, 0)),
            pl.BlockSpec((_B, _B), lambda i: (0, 0)),
            pl.BlockSpec((_B, _B), lambda i: (0, 0)),
        ],
        out_shape=[
            jax.ShapeDtypeStruct((_B, _N, _H), jnp.bfloat16),
            jax.ShapeDtypeStruct((_B, _N, _H), jnp.bfloat16),
            jax.ShapeDtypeStruct((_B, _B), f32),
            jax.ShapeDtypeStruct((_B, _B), f32),
        ],
        scratch_shapes=[
            pltpu.VMEM((_B, _H), f32),
            pltpu.VMEM((_B, _H), f32),
        ],
        compiler_params=pltpu.CompilerParams(
            dimension_semantics=("arbitrary",)),
    )(A_batch, feature,
      W_sc0, b2(b_sc0), W_sc1, b2(b_sc1), W_sc2, b2(b_sc2),
      W_fc0, b2(b_fc0), W_fc1, b2(b_fc1), W_fc2, b2(b_fc2),
      m1, m2)

    sc_cp = pltpu.CompilerParams()
    if "needs_layout_passes" in pltpu.CompilerParams.__dataclass_fields__:
        sc_cp = dataclasses.replace(sc_cp, needs_layout_passes=False)
    sc_mesh = plsc.VectorSubcoreMesh(core_axis_name="c", subcore_axis_name="s")
    sc_topk = pl.kernel(
        _sc_topk_body,
        out_type=[
            jax.ShapeDtypeStruct((_B, _NBR_PAD), jnp.int32),
            jax.ShapeDtypeStruct((_B, _NBR_PAD), jnp.int32),
        ],
        mesh=sc_mesh,
        scratch_types=[
            pltpu.VMEM((_B,), f32),
            pltpu.VMEM((_NBR_PAD,), jnp.int32),
            pltpu.SemaphoreType.DMA,
        ],
        compiler_params=sc_cp,
    )
    nbr1, nbr2 = sc_topk(dm1, dm2)

    m1v = no_sc_idx.astype(jnp.int32)
    m2v = no_fc_idx.astype(jnp.int32)

    def nbr_map(j, branch, g):
        if branch == 1:
            return lambda b, n1, n2, mm1, mm2: (
                jnp.where(mm1[b * _G3 + g] != 0,
                          n1[b * _G3 + g, j], b * _G3 + g), 0, 0)
        return lambda b, n1, n2, mm1, mm2: (
            jnp.where(mm2[b * _G3 + g] != 0,
                      n2[b * _G3 + g, j], b * _G3 + g), 0, 0)

    grid_spec = pltpu.PrefetchScalarGridSpec(
        num_scalar_prefetch=4,
        grid=(_B // _G3,),
        in_specs=[
            *[pl.BlockSpec((1, _N, _H), nbr_map(j, 1, g))
              for g in range(_G3) for j in range(_K)],
            *[pl.BlockSpec((1, _N, _H), nbr_map(j, 2, g))
              for g in range(_G3) for j in range(_K)],
            pl.BlockSpec((_H, _H), lambda *_: (0, 0)),
            pl.BlockSpec((_H, _H), lambda *_: (0, 0)),
            pl.BlockSpec((1, _H), lambda *_: (0, 0)),
            pl.BlockSpec((1, _H), lambda *_: (0, 0)),
            pl.BlockSpec((_H, 1), lambda *_: (0, 0)),
            pl.BlockSpec((_H, _OUT), lambda *_: (0, 0)),
            pl.BlockSpec((1, _OUT), lambda *_: (0, 0)),
        ],
        out_specs=pl.BlockSpec((_B, _OUT), lambda b, *_: (0, 0)),
        scratch_shapes=[
            pltpu.VMEM((_H, 1), f32),
            pltpu.VMEM((_H, 1), f32),
            pltpu.VMEM((1, 1), f32),
            pltpu.VMEM((1, 1), f32),
        ],
    )
    out = pl.pallas_call(
        _fuse_body,
        grid_spec=grid_spec,
        out_shape=jax.ShapeDtypeStruct((_B, _OUT), f32),
        compiler_params=pltpu.CompilerParams(
            dimension_semantics=("arbitrary",)),
    )(nbr1, nbr2, m1v, m2v,
      *([es] * (_G3 * _K)), *([ef] * (_G3 * _K)),
      w1_w, w2_w, b2(w1_b), b2(w2_b), attention, out_w, b2(out_b))
    return out
